# baseline (speedup 1.0000x reference)
import math

import jax
import jax.numpy as jnp
from jax.experimental import pallas as pl
from jax.experimental.pallas import tpu as pltpu

_CH = 2048
_NEG = -1e30


def _attn_kernel(starts_ref, ends_ref, enc_ref, wq_ref, bq_ref, wk_ref,
                 bk_ref, wv_ref, bv_ref, soc_hbm, out_ref, bufs, sems):
    b, d = out_ref.shape
    t = soc_hbm.shape[0]
    nch = t // _CH

    def copy(i):
        return pltpu.make_async_copy(
            soc_hbm.at[pl.ds(i * _CH, _CH), :], bufs.at[i], sems.at[i])

    for i in range(nch):
        copy(i).start()

    _t = (((1,), (1,)), ((), ()))
    q = jax.lax.dot_general(enc_ref[...], wq_ref[...], _t,
                            preferred_element_type=jnp.float32) + bq_ref[...]
    q = (jnp.maximum(q, 0.0) * (1.0 / math.sqrt(d))).astype(jnp.bfloat16)

    starts = starts_ref[...]
    ends = ends_ref[...]
    wk, bk = wk_ref[...].astype(jnp.bfloat16), bk_ref[...]
    wv, bv = wv_ref[...].astype(jnp.bfloat16), bv_ref[...]

    m = jnp.full((b, 1), _NEG, jnp.float32)
    s = jnp.zeros((b, 1), jnp.float32)
    acc = jnp.zeros((b, d), jnp.float32)

    for j in range(nch):
        copy(j).wait()
        tok = bufs[j].astype(jnp.bfloat16)
        k = jnp.maximum(jax.lax.dot_general(
            tok, wk, _t, preferred_element_type=jnp.float32) + bk,
            0.0).astype(jnp.bfloat16)
        v = jnp.maximum(jax.lax.dot_general(
            tok, wv, _t, preferred_element_type=jnp.float32) + bv,
            0.0).astype(jnp.bfloat16)

        logits = jax.lax.dot_general(
            q, k, _t, preferred_element_type=jnp.float32)
        col = j * _CH + jax.lax.broadcasted_iota(jnp.int32, (b, _CH), 1)
        mask = (col >= starts) & (col < ends)
        logits = jnp.where(mask, logits, _NEG)

        m_new = jnp.maximum(m, jnp.max(logits, axis=1, keepdims=True))
        alpha = jnp.exp(m - m_new)
        p = jnp.exp(logits - m_new)
        s = s * alpha + jnp.sum(p, axis=1, keepdims=True)
        acc = acc * alpha + jnp.dot(p.astype(jnp.bfloat16), v,
                                    preferred_element_type=jnp.float32)
        m = m_new

    out_ref[...] = acc / s


def kernel(enc_hidden, social_ht, neighbors_idx_start, neighbors_idx_end,
           Wq, bq, Wk, bk, Wv, bv):
    b, d = enc_hidden.shape
    t = social_ht.shape[0]
    nch = t // _CH

    starts = neighbors_idx_start.astype(jnp.int32).reshape(b, 1)
    ends = neighbors_idx_end.astype(jnp.int32).reshape(b, 1)

    vmem = pl.BlockSpec(memory_space=pltpu.MemorySpace.VMEM)
    out = pl.pallas_call(
        _attn_kernel,
        in_specs=[vmem, vmem, vmem, vmem, vmem, vmem, vmem, vmem, vmem,
                  pl.BlockSpec(memory_space=pltpu.MemorySpace.HBM)],
        out_specs=vmem,
        out_shape=jax.ShapeDtypeStruct((b, d), jnp.float32),
        scratch_shapes=[
            pltpu.VMEM((nch, _CH, d), jnp.float32),
            pltpu.SemaphoreType.DMA((nch,)),
        ],
    )(starts, ends, enc_hidden,
      Wq, bq.reshape(1, d),
      Wk, bk.reshape(1, d),
      Wv, bv.reshape(1, d), social_ht)
    return out


# speedup vs baseline: 1.2335x; 1.2335x over previous
import math

import jax
import jax.numpy as jnp
from jax.experimental import pallas as pl
from jax.experimental.pallas import tpu as pltpu

_CH = 8192
_DSUB = 2048
_LANES = 2
_NEG = -1e30


def _attn_kernel(starts_ref, ends_ref, enc_ref, wq_ref, bq_ref, wk_ref,
                 bk_ref, wv_ref, bv_ref, soc_hbm, out_ref, bufs, sems):
    b, d = out_ref.shape
    t = soc_hbm.shape[0]
    nch = t // _CH

    nsub = _CH // _DSUB

    def copy(i, u):
        return pltpu.make_async_copy(
            soc_hbm.at[pl.ds(i * _CH + u * _DSUB, _DSUB), :],
            bufs.at[i, pl.ds(u * _DSUB, _DSUB), :], sems.at[i, u])

    for i in range(nch):
        for u in range(nsub):
            copy(i, u).start()

    _t = (((1,), (1,)), ((), ()))
    q = jax.lax.dot_general(enc_ref[...], wq_ref[...], _t,
                            preferred_element_type=jnp.float32) + bq_ref[...]
    q = (jnp.maximum(q, 0.0) * (1.0 / math.sqrt(d))).astype(jnp.bfloat16)

    starts = starts_ref[...]
    ends = ends_ref[...]
    wk, bk = wk_ref[...].astype(jnp.bfloat16), bk_ref[...]
    wv, bv = wv_ref[...].astype(jnp.bfloat16), bv_ref[...]

    ms = [jnp.full((b, 1), _NEG, jnp.float32) for _ in range(_LANES)]
    ss = [jnp.zeros((b, 1), jnp.float32) for _ in range(_LANES)]
    accs = [jnp.zeros((b, d), jnp.float32) for _ in range(_LANES)]

    for j in range(nch):
        c = j % _LANES
        for u in range(nsub):
            copy(j, u).wait()
        tok = bufs[j].astype(jnp.bfloat16)
        k = jnp.maximum(jax.lax.dot_general(
            tok, wk, _t, preferred_element_type=jnp.float32) + bk,
            0.0).astype(jnp.bfloat16)
        v = jnp.maximum(jax.lax.dot_general(
            tok, wv, _t, preferred_element_type=jnp.float32) + bv,
            0.0).astype(jnp.bfloat16)

        logits = jax.lax.dot_general(
            q, k, _t, preferred_element_type=jnp.float32)
        col = j * _CH + jax.lax.broadcasted_iota(jnp.int32, (b, _CH), 1)
        mask = (col >= starts) & (col < ends)
        logits = jnp.where(mask, logits, _NEG)

        m_new = jnp.maximum(ms[c], jnp.max(logits, axis=1, keepdims=True))
        alpha = jnp.exp(ms[c] - m_new)
        p = jnp.exp(logits - m_new)
        ss[c] = ss[c] * alpha + jnp.sum(p, axis=1, keepdims=True)
        accs[c] = accs[c] * alpha + jnp.dot(p.astype(jnp.bfloat16), v,
                                            preferred_element_type=jnp.float32)
        ms[c] = m_new

    m, s, acc = ms[0], ss[0], accs[0]
    for c in range(1, _LANES):
        m2 = jnp.maximum(m, ms[c])
        w1 = jnp.exp(m - m2)
        w2 = jnp.exp(ms[c] - m2)
        s = s * w1 + ss[c] * w2
        acc = acc * w1 + accs[c] * w2
        m = m2

    out_ref[...] = acc / s


def kernel(enc_hidden, social_ht, neighbors_idx_start, neighbors_idx_end,
           Wq, bq, Wk, bk, Wv, bv):
    b, d = enc_hidden.shape
    t = social_ht.shape[0]
    nch = t // _CH

    starts = neighbors_idx_start.astype(jnp.int32).reshape(b, 1)
    ends = neighbors_idx_end.astype(jnp.int32).reshape(b, 1)

    vmem = pl.BlockSpec(memory_space=pltpu.MemorySpace.VMEM)
    out = pl.pallas_call(
        _attn_kernel,
        in_specs=[vmem, vmem, vmem, vmem, vmem, vmem, vmem, vmem, vmem,
                  pl.BlockSpec(memory_space=pltpu.MemorySpace.HBM)],
        out_specs=vmem,
        out_shape=jax.ShapeDtypeStruct((b, d), jnp.float32),
        scratch_shapes=[
            pltpu.VMEM((nch, _CH, d), jnp.float32),
            pltpu.SemaphoreType.DMA((nch, _CH // _DSUB)),
        ],
    )(starts, ends, enc_hidden,
      Wq, bq.reshape(1, d),
      Wk, bk.reshape(1, d),
      Wv, bv.reshape(1, d), social_ht)
    return out


# speedup vs baseline: 1.4576x; 1.1817x over previous
import math

import jax
import jax.numpy as jnp
from jax.experimental import pallas as pl
from jax.experimental.pallas import tpu as pltpu

_CH = 8192
_DSUB = 2048
_LANES = 2
_NEG = -1e30


def _attn_kernel(starts_ref, ends_ref, enc_ref, wq_ref, bq_ref, wk_ref,
                 bk_ref, wv_ref, bv_ref, soc_hbm, out_ref, bufs, sems):
    b, d = out_ref.shape
    t = soc_hbm.shape[0]
    nch = t // _CH

    nsub = _CH // _DSUB

    def copy(i, u):
        return pltpu.make_async_copy(
            soc_hbm.at[pl.ds(i * _CH + u * _DSUB, _DSUB), :],
            bufs.at[i, pl.ds(u * _DSUB, _DSUB), :], sems.at[i, u])

    for i in range(nch):
        for u in range(nsub):
            copy(i, u).start()

    _t = (((1,), (1,)), ((), ()))
    q = jax.lax.dot_general(enc_ref[...], wq_ref[...], _t,
                            preferred_element_type=jnp.float32) + bq_ref[...][None, :]
    q = (jnp.maximum(q, 0.0) * (1.0 / math.sqrt(d))).astype(jnp.bfloat16)

    starts = starts_ref[...].reshape(b, 1)
    ends = ends_ref[...].reshape(b, 1)
    wk, bk = wk_ref[...].astype(jnp.bfloat16), bk_ref[...][None, :]
    wv, bv = wv_ref[...].astype(jnp.bfloat16), bv_ref[...][None, :]

    ms = [jnp.full((b, 1), _NEG, jnp.float32) for _ in range(_LANES)]
    ss = [jnp.zeros((b, 1), jnp.float32) for _ in range(_LANES)]
    accs = [jnp.zeros((b, d), jnp.float32) for _ in range(_LANES)]

    for j in range(nch):
        c = j % _LANES
        for u in range(nsub):
            copy(j, u).wait()
        tok = bufs[j].astype(jnp.bfloat16)
        k = jnp.maximum(jax.lax.dot_general(
            tok, wk, _t, preferred_element_type=jnp.float32) + bk,
            0.0).astype(jnp.bfloat16)
        v = jnp.maximum(jax.lax.dot_general(
            tok, wv, _t, preferred_element_type=jnp.float32) + bv,
            0.0).astype(jnp.bfloat16)

        logits = jax.lax.dot_general(
            q, k, _t, preferred_element_type=jnp.float32)
        col = j * _CH + jax.lax.broadcasted_iota(jnp.int32, (b, _CH), 1)
        mask = (col >= starts) & (col < ends)
        logits = jnp.where(mask, logits, _NEG)

        m_new = jnp.maximum(ms[c], jnp.max(logits, axis=1, keepdims=True))
        alpha = jnp.exp(ms[c] - m_new)
        p = jnp.exp(logits - m_new)
        ss[c] = ss[c] * alpha + jnp.sum(p, axis=1, keepdims=True)
        accs[c] = accs[c] * alpha + jnp.dot(p.astype(jnp.bfloat16), v,
                                            preferred_element_type=jnp.float32)
        ms[c] = m_new

    m, s, acc = ms[0], ss[0], accs[0]
    for c in range(1, _LANES):
        m2 = jnp.maximum(m, ms[c])
        w1 = jnp.exp(m - m2)
        w2 = jnp.exp(ms[c] - m2)
        s = s * w1 + ss[c] * w2
        acc = acc * w1 + accs[c] * w2
        m = m2

    out_ref[...] = acc / s


def kernel(enc_hidden, social_ht, neighbors_idx_start, neighbors_idx_end,
           Wq, bq, Wk, bk, Wv, bv):
    b, d = enc_hidden.shape
    t = social_ht.shape[0]
    nch = t // _CH

    starts = neighbors_idx_start.astype(jnp.int32)
    ends = neighbors_idx_end.astype(jnp.int32)

    vmem = pl.BlockSpec(memory_space=pltpu.MemorySpace.VMEM)
    out = pl.pallas_call(
        _attn_kernel,
        in_specs=[vmem, vmem, vmem, vmem, vmem, vmem, vmem, vmem, vmem,
                  pl.BlockSpec(memory_space=pltpu.MemorySpace.HBM)],
        out_specs=vmem,
        out_shape=jax.ShapeDtypeStruct((b, d), jnp.float32),
        scratch_shapes=[
            pltpu.VMEM((nch, _CH, d), jnp.float32),
            pltpu.SemaphoreType.DMA((nch, _CH // _DSUB)),
        ],
    )(starts, ends, enc_hidden, Wq, bq, Wk, bk, Wv, bv, social_ht)
    return out
